# Initial kernel scaffold; baseline (speedup 1.0000x reference)
import jax
import jax.numpy as jnp
from jax.experimental import pallas as pl


def kernel(inputs):
    raise NotImplementedError("write your pallas kernel here")


# speedup vs baseline: 19.3936x; 19.3936x over previous
import jax
import jax.numpy as jnp
from jax import lax
from jax.experimental import pallas as pl
from jax.experimental.pallas import tpu as pltpu
from jax.experimental.pallas import tpu_sc as plsc

NC, NSUB, L = 2, 16, 16
NW = NC * NSUB
NPAIR = 16 * 32
PAIRS_PER_W = NPAIR // NW
NCH = 4
WORDS = 8192 * NCH
K = 16


def _sortd(v):
    k, _ = plsc.sort_key_val(v, v, descending=True)
    return k


def _merge(a, b):
    return _sortd(jnp.maximum(a, lax.rev(b, (0,))))


def _sc_body(in_hbm, out_hbm, buf, outbuf, sem):
    del sem
    wid = lax.axis_index("s") * NC + lax.axis_index("c")
    iota = lax.iota(jnp.int32, L)
    neg_inf = jnp.full((L,), -jnp.inf, dtype=jnp.float32)

    def pair_body(i, carry):
        p = wid * PAIRS_PER_W + i
        pltpu.sync_copy(in_hbm.at[p], buf)
        for c in range(NCH):
            iota4c = iota * NCH + c

            def chunk_body(m, t):
                base = iota4c + m * 1024
                s = []
                for u in range(16):
                    v = plsc.load_gather(buf, [base + 64 * u])
                    s.append(_sortd(v))
                while len(s) > 1:
                    s = [_merge(s[2 * j], s[2 * j + 1])
                         for j in range(len(s) // 2)]
                return _merge(t, s[0])

            top = lax.fori_loop(0, 32, chunk_body, neg_inf)
            outbuf[pl.ds(c * K, K)] = top
        pltpu.sync_copy(outbuf, out_hbm.at[p])
        return carry

    lax.fori_loop(0, PAIRS_PER_W, pair_body, 0)


def kernel(inputs):
    flat = inputs.reshape(NPAIR, WORDS)
    mesh = plsc.VectorSubcoreMesh(
        core_axis_name="c", subcore_axis_name="s",
        num_cores=NC, num_subcores=NSUB)
    out = pl.kernel(
        _sc_body,
        out_type=jax.ShapeDtypeStruct((NPAIR, NCH * K), jnp.float32),
        mesh=mesh,
        scratch_types=[
            pltpu.VMEM((WORDS,), jnp.float32),
            pltpu.VMEM((NCH * K,), jnp.float32),
            pltpu.SemaphoreType.DMA,
        ],
        compiler_params=pltpu.CompilerParams(needs_layout_passes=False),
    )(flat)
    return out.reshape(16, 32, NCH * K)


# speedup vs baseline: 20.7903x; 1.0720x over previous
import jax
import jax.numpy as jnp
from jax import lax
from jax.experimental import pallas as pl
from jax.experimental.pallas import tpu as pltpu
from jax.experimental.pallas import tpu_sc as plsc

NC, NSUB, L = 2, 16, 16
NW = NC * NSUB
NPAIR = 16 * 32
PAIRS_PER_W = NPAIR // NW
NCH = 4
WORDS = 8192 * NCH
K = 16
NACC = 16
NGRP = NACC * L
GSZ = 8192 // NGRP


def _sortd(v):
    k, _ = plsc.sort_key_val(v, v, descending=True)
    return k


def _merge(a, b):
    return _sortd(jnp.maximum(a, lax.rev(b, (0,))))


def _tree_top16(vs):
    s = [_sortd(v) for v in vs]
    while len(s) > 1:
        if len(s) % 2:
            s.append(None)
        s = [s[2 * j] if s[2 * j + 1] is None else _merge(s[2 * j], s[2 * j + 1])
             for j in range(len(s) // 2)]
    return s[0]


def _sc_body(in_hbm, out_hbm, buf, glist, outbuf, sem):
    wid = lax.axis_index("s") * NC + lax.axis_index("c")
    iota = lax.iota(jnp.int32, L)
    neg_inf = jnp.full((L,), -jnp.inf, dtype=jnp.float32)
    p0 = wid * PAIRS_PER_W

    pltpu.async_copy(in_hbm.at[p0], buf.at[pl.ds(0, WORDS)], sem)

    def pair_body(i, carry):
        p = p0 + i
        sel = lax.rem(i, 2)
        bbase = sel * WORDS
        pltpu.make_async_copy(in_hbm.at[p], buf.at[pl.ds(bbase, WORDS)],
                              sem).wait()

        @pl.when(i + 1 < PAIRS_PER_W)
        def _prefetch():
            pltpu.async_copy(in_hbm.at[p + 1],
                             buf.at[pl.ds((1 - sel) * WORDS, WORDS)], sem)

        flat = buf
        for c in range(NCH):
            iota4c = iota * NCH + c + bbase

            def chunk_body(m, accs):
                base = iota4c + m * (NGRP * NCH)
                return tuple(
                    jnp.maximum(a, plsc.load_gather(flat, [base + (NCH * L) * u]))
                    for u, a in enumerate(accs))

            accs = lax.fori_loop(0, GSZ, chunk_body, (neg_inf,) * NACC)

            t = jnp.min(_tree_top16(list(accs)))

            cnt = jnp.int32(0)
            for u in range(NACC):
                mask = accs[u] >= t
                plsc.store_compressed(glist.at[pl.ds(cnt, L)], iota + L * u,
                                      mask=mask)
                cnt = cnt + plsc.all_reduce_population_count(mask)[0]
            glist[pl.ds(cnt, L)] = jnp.zeros((L,), jnp.int32)

            def block_body(b, top):
                gids = glist[pl.ds(L * b, L)]
                valid = (L * b + iota) < cnt
                gbase = gids * NCH + c + bbase
                vs = []
                for m in range(GSZ):
                    v = plsc.load_gather(flat, [gbase + m * (NGRP * NCH)])
                    vs.append(jnp.where(valid, v, neg_inf))
                return _merge(top, _tree_top16(vs))

            nblk = (cnt + (L - 1)) // L
            top = lax.fori_loop(0, nblk, block_body, neg_inf)
            outbuf[pl.ds(c * K, K)] = top
        pltpu.sync_copy(outbuf, out_hbm.at[p])
        return carry

    lax.fori_loop(0, PAIRS_PER_W, pair_body, 0)


def kernel(inputs):
    flat = inputs.reshape(NPAIR, WORDS)
    mesh = plsc.VectorSubcoreMesh(
        core_axis_name="c", subcore_axis_name="s",
        num_cores=NC, num_subcores=NSUB)
    out = pl.kernel(
        _sc_body,
        out_type=jax.ShapeDtypeStruct((NPAIR, NCH * K), jnp.float32),
        mesh=mesh,
        scratch_types=[
            pltpu.VMEM((2 * WORDS,), jnp.float32),
            pltpu.VMEM((NGRP + L,), jnp.int32),
            pltpu.VMEM((NCH * K,), jnp.float32),
            pltpu.SemaphoreType.DMA,
        ],
        compiler_params=pltpu.CompilerParams(needs_layout_passes=False),
    )(flat)
    return out.reshape(16, 32, NCH * K)


# speedup vs baseline: 21.3623x; 1.0275x over previous
import jax
import jax.numpy as jnp
from jax import lax
from jax.experimental import pallas as pl
from jax.experimental.pallas import tpu as pltpu
from jax.experimental.pallas import tpu_sc as plsc

NC, NSUB, L = 2, 16, 16
NW = NC * NSUB
NPAIR = 16 * 32
PAIRS_PER_W = NPAIR // NW
NCH = 4
WORDS = 8192 * NCH
K = 16
NBLK = WORDS // (8 * L)
NGRP = NBLK * NCH
GVECS = NGRP // L


def _sortd(v):
    k, _ = plsc.sort_key_val(v, v, descending=True)
    return k


def _merge(a, b):
    return _sortd(jnp.maximum(a, lax.rev(b, (0,))))


def _sortd_kv(k, v):
    sk, sv = plsc.sort_key_val(k, v, descending=True)
    return sk, sv


def _merge_kv(ak, av, bk, bv):
    rk, rv = lax.rev(bk, (0,)), lax.rev(bv, (0,))
    m = ak >= rk
    return _sortd_kv(jnp.where(m, ak, rk), jnp.where(m, av, rv))


def _tree_top16(vs):
    s = list(vs)
    while len(s) > 1:
        if len(s) % 2:
            s.append(None)
        s = [s[2 * j] if s[2 * j + 1] is None else _merge(s[2 * j], s[2 * j + 1])
             for j in range(len(s) // 2)]
    return s[0]


def _sc_body(in_hbm, out_hbm, buf, gmax, outbuf, sem):
    wid = lax.axis_index("s") * NC + lax.axis_index("c")
    iota = lax.iota(jnp.int32, L)
    p0 = wid * PAIRS_PER_W

    pltpu.async_copy(in_hbm.at[p0], buf.at[pl.ds(0, WORDS)], sem)

    def pair_body(i, carry):
        p = p0 + i
        sel = lax.rem(i, 2)
        bbase = sel * WORDS
        pltpu.make_async_copy(in_hbm.at[p], buf.at[pl.ds(bbase, WORDS)],
                              sem).wait()

        @pl.when(i + 1 < PAIRS_PER_W)
        def _prefetch():
            pltpu.async_copy(in_hbm.at[p + 1],
                             buf.at[pl.ds((1 - sel) * WORDS, WORDS)], sem)

        def p1_body(g, carry_):
            base = bbase + g * (16 * L)
            for half in range(2):
                hb = base + half * (8 * L)
                acc = buf[pl.ds(hb, L)]
                for m in range(1, 8):
                    acc = jnp.maximum(acc, buf[pl.ds(hb + m * L, L)])
                gmax[pl.ds((2 * g + half) * L, L)] = acc
            return carry_

        lax.fori_loop(0, NBLK // 2, p1_body, 0)

        for c in range(NCH):
            idx0 = iota * NCH + c

            def p2_body(n, tkv):
                tk, tv = tkv
                sub = []
                for j in range(4):
                    vbase = (4 * n + j) * L
                    keys = plsc.load_gather(gmax, [idx0 + NCH * vbase])
                    sub.append(_sortd_kv(keys, vbase + iota))
                (k0, v0), (k1, v1), (k2, v2), (k3, v3) = sub
                ka, va = _merge_kv(k0, v0, k1, v1)
                kb, vb = _merge_kv(k2, v2, k3, v3)
                kc, vc = _merge_kv(ka, va, kb, vb)
                return _merge_kv(tk, tv, kc, vc)

            neg_inf = jnp.full((L,), -jnp.inf, dtype=jnp.float32)
            _, gsel = lax.fori_loop(0, GVECS // 4, p2_body,
                                    (neg_inf, jnp.zeros((L,), jnp.int32)))

            w = bbase + ((gsel >> 2) << 7) + ((gsel & 3) << 2) + c
            vs = [_sortd(plsc.load_gather(buf, [w + m * L])) for m in range(8)]
            outbuf[pl.ds(c * K, K)] = _tree_top16(vs)

        pltpu.sync_copy(outbuf, out_hbm.at[p])
        return carry

    lax.fori_loop(0, PAIRS_PER_W, pair_body, 0)


def kernel(inputs):
    flat = inputs.reshape(NPAIR, WORDS)
    mesh = plsc.VectorSubcoreMesh(
        core_axis_name="c", subcore_axis_name="s",
        num_cores=NC, num_subcores=NSUB)
    out = pl.kernel(
        _sc_body,
        out_type=jax.ShapeDtypeStruct((NPAIR, NCH * K), jnp.float32),
        mesh=mesh,
        scratch_types=[
            pltpu.VMEM((2 * WORDS,), jnp.float32),
            pltpu.VMEM((NGRP * NCH,), jnp.float32),
            pltpu.VMEM((NCH * K,), jnp.float32),
            pltpu.SemaphoreType.DMA,
        ],
        compiler_params=pltpu.CompilerParams(needs_layout_passes=False),
    )(flat)
    return out.reshape(16, 32, NCH * K)
